# Initial kernel scaffold; baseline (speedup 1.0000x reference)
import jax
import jax.numpy as jnp
from jax.experimental import pallas as pl


def kernel(xi, Gi2i, xj, Gj2i, W1, b1, W2, b2):
    raise NotImplementedError("write your pallas kernel here")


# speedup vs baseline: 1.1775x; 1.1775x over previous
import jax
import jax.numpy as jnp
from jax.experimental import pallas as pl
from jax.experimental.pallas import tpu as pltpu

N_I_ = 4096
N_J_ = 8192
CH = 256
M_BLK = 256


def _body(xi_ref, gii_ref, xj_ref, gji_ref, w1_ref, b1_ref, w2_ref, b2_ref,
          out_ref, yi_ref, yj_ref):
    i = pl.program_id(0)

    @pl.when(i == 0)
    def _prologue():
        yi = jnp.dot(xi_ref[...].astype(jnp.bfloat16),
                     w1_ref[...].astype(jnp.bfloat16),
                     preferred_element_type=jnp.float32) + b1_ref[...]
        yi_ref[...] = yi.astype(jnp.bfloat16)
        yj = jnp.dot(xj_ref[...].astype(jnp.bfloat16),
                     w2_ref[...].astype(jnp.bfloat16),
                     preferred_element_type=jnp.float32) + b2_ref[...]
        yj_ref[...] = yj.astype(jnp.bfloat16)

    acc = jnp.dot(gii_ref[...].astype(jnp.bfloat16), yi_ref[...],
                  preferred_element_type=jnp.float32)
    acc = acc + jnp.dot(gji_ref[...].astype(jnp.bfloat16), yj_ref[...],
                        preferred_element_type=jnp.float32)
    out_ref[...] = jnp.maximum(acc, 0.0)


def kernel(xi, Gi2i, xj, Gj2i, W1, b1, W2, b2):
    n_i = Gi2i.shape[0]
    n_j = xj.shape[0]
    grid = (n_i // M_BLK,)
    return pl.pallas_call(
        _body,
        grid=grid,
        in_specs=[
            pl.BlockSpec((n_i, CH), lambda i: (0, 0)),
            pl.BlockSpec((M_BLK, n_i), lambda i: (i, 0)),
            pl.BlockSpec((n_j, CH), lambda i: (0, 0)),
            pl.BlockSpec((M_BLK, n_j), lambda i: (i, 0)),
            pl.BlockSpec((CH, CH), lambda i: (0, 0)),
            pl.BlockSpec((1, CH), lambda i: (0, 0)),
            pl.BlockSpec((CH, CH), lambda i: (0, 0)),
            pl.BlockSpec((1, CH), lambda i: (0, 0)),
        ],
        out_specs=pl.BlockSpec((M_BLK, CH), lambda i: (i, 0)),
        out_shape=jax.ShapeDtypeStruct((n_i, CH), jnp.float32),
        scratch_shapes=[
            pltpu.VMEM((n_i, CH), jnp.bfloat16),
            pltpu.VMEM((n_j, CH), jnp.bfloat16),
        ],
    )(xi, Gi2i, xj, Gj2i, W1, b1.reshape(1, CH), W2, b2.reshape(1, CH))
